# baseline (speedup 1.0000x reference)
import functools

import jax
import jax.numpy as jnp
from jax import lax
from jax.experimental import pallas as pl
from jax.experimental.pallas import tpu as pltpu
from jax.experimental.pallas import tpu_sc as plsc


def _make_sc_gather(V, D, B):
    info = plsc.get_sparse_core_info()
    NC, NS = info.num_cores, info.num_subcores
    NW = NC * NS
    assert B % NW == 0
    b_per_w = B // NW
    CH = 128 if b_per_w % 128 == 0 else b_per_w
    n_ch = b_per_w // CH
    mesh = plsc.VectorSubcoreMesh(core_axis_name="c", subcore_axis_name="s")

    @functools.partial(
        pl.kernel,
        mesh=mesh,
        out_type=jax.ShapeDtypeStruct((B, D), jnp.float32),
        scratch_types=[
            pltpu.VMEM((b_per_w,), jnp.int32),
            pltpu.VMEM((b_per_w, D), jnp.float32),
            pltpu.SemaphoreType.DMA,
        ],
    )
    def sc_gather(table_hbm, idx_hbm, out_hbm, idx_v, rows_v, sem):
        wid = lax.axis_index("s") * NC + lax.axis_index("c")
        base = wid * b_per_w
        pltpu.sync_copy(idx_hbm.at[pl.ds(base, b_per_w)], idx_v)
        copies = []
        for j in range(n_ch):
            copies.append(pltpu.async_copy(
                table_hbm.at[idx_v.at[pl.ds(j * CH, CH)]],
                rows_v.at[pl.ds(j * CH, CH)],
                sem,
            ))
        for c in copies:
            c.wait()
        pltpu.sync_copy(rows_v, out_hbm.at[pl.ds(base, b_per_w)])

    return sc_gather


def _mlp_body(x_ref, w1_ref, b1_ref, w2_ref, b2_ref, o_ref):
    nt, td = b2_ref.shape
    x = x_ref[...].astype(jnp.bfloat16)
    h = jnp.dot(x, w1_ref[...], preferred_element_type=jnp.float32) + b1_ref[...]
    h = (h * jax.nn.sigmoid(h)).astype(jnp.bfloat16)
    o = jnp.dot(h, w2_ref[...], preferred_element_type=jnp.float32)
    o_ref[...] = o.reshape(o.shape[0], nt, td) + b2_ref[...]


def _mlp_body_alias(x_ref, w1_ref, b1_ref, w2_ref, b2_ref, _, o_ref):
    _mlp_body(x_ref, w1_ref, b1_ref, w2_ref, b2_ref, o_ref)


def _tc_mlp_chunk(x_chunk, W1, b1, W2, bias2, out_buf, B, blk0, blk):
    CB, D = x_chunk.shape
    H = W1.shape[1]
    O = W2.shape[1]
    NT = bias2.shape[0]
    TD = O // NT
    grid = (CB // blk,)
    in_specs = [
        pl.BlockSpec((blk, D), lambda i: (i, 0)),
        pl.BlockSpec((D, H), lambda i: (0, 0)),
        pl.BlockSpec((1, H), lambda i: (0, 0)),
        pl.BlockSpec((H, O), lambda i: (0, 0)),
        pl.BlockSpec((NT, TD), lambda i: (0, 0)),
    ]
    args = (x_chunk, W1, b1, W2, bias2)
    body = _mlp_body
    aliases = {}
    if out_buf is not None:
        in_specs.append(pl.BlockSpec(memory_space=pl.ANY))
        args = args + (out_buf,)
        body = _mlp_body_alias
        aliases = {5: 0}
    return pl.pallas_call(
        body,
        grid=grid,
        in_specs=in_specs,
        out_specs=pl.BlockSpec((blk, NT, TD), lambda i: (i + blk0, 0, 0)),
        out_shape=jax.ShapeDtypeStruct((B, NT, TD), jnp.float32),
        input_output_aliases=aliases,
    )(*args)


def kernel(labels, table, W1, b1, W2, b2, pos):
    B = labels.shape[0]
    V, D = table.shape
    NT, _ = pos.shape
    O = W2.shape[1]
    idx = labels.astype(jnp.int32)
    bias2 = b2.reshape(NT, D) + pos
    W1b = W1.astype(jnp.bfloat16)
    W2b = W2.astype(jnp.bfloat16)
    b1r = b1[None, :]

    NCHUNK = 2
    BLK = 2048
    CB = B // NCHUNK
    gather = _make_sc_gather(V, D, CB)
    xs = [gather(table, lax.slice(idx, (k * CB,), ((k + 1) * CB,)))
          for k in range(NCHUNK)]

    out = _tc_mlp_chunk(xs[0], W1b, b1r, W2b, bias2, None, B, 0, BLK)
    for k in range(1, NCHUNK):
        out = _tc_mlp_chunk(xs[k], W1b, b1r, W2b, bias2, out, B,
                            k * (CB // BLK), BLK)
    return out


# speedup vs baseline: 1.0537x; 1.0537x over previous
import functools

import jax
import jax.numpy as jnp
from jax import lax
from jax.experimental import pallas as pl
from jax.experimental.pallas import tpu as pltpu
from jax.experimental.pallas import tpu_sc as plsc


def _make_sc_gather(V, D, B):
    info = plsc.get_sparse_core_info()
    NC, NS = info.num_cores, info.num_subcores
    NW = NC * NS
    assert B % NW == 0
    b_per_w = B // NW
    CH = 128 if b_per_w % 128 == 0 else b_per_w
    n_ch = b_per_w // CH
    mesh = plsc.VectorSubcoreMesh(core_axis_name="c", subcore_axis_name="s")

    @functools.partial(
        pl.kernel,
        mesh=mesh,
        out_type=jax.ShapeDtypeStruct((B, D), jnp.float32),
        scratch_types=[
            pltpu.VMEM((b_per_w,), jnp.int32),
            pltpu.VMEM((b_per_w, D), jnp.float32),
            pltpu.SemaphoreType.DMA,
        ],
    )
    def sc_gather(table_hbm, idx_hbm, out_hbm, idx_v, rows_v, sem):
        wid = lax.axis_index("s") * NC + lax.axis_index("c")
        base = wid * b_per_w
        pltpu.sync_copy(idx_hbm.at[pl.ds(base, b_per_w)], idx_v)
        copies = []
        for j in range(n_ch):
            copies.append(pltpu.async_copy(
                table_hbm.at[idx_v.at[pl.ds(j * CH, CH)]],
                rows_v.at[pl.ds(j * CH, CH)],
                sem,
            ))
        for c in copies:
            c.wait()
        pltpu.sync_copy(rows_v, out_hbm.at[pl.ds(base, b_per_w)])

    return sc_gather


def _mlp_body(x_ref, w1_ref, b1_ref, w2_ref, b2_ref, o_ref):
    nt, td = b2_ref.shape
    x = x_ref[...].astype(jnp.bfloat16)
    h = jnp.dot(x, w1_ref[...], preferred_element_type=jnp.float32) + b1_ref[...]
    h = (h * jax.nn.sigmoid(h)).astype(jnp.bfloat16)
    o = jnp.dot(h, w2_ref[...], preferred_element_type=jnp.float32)
    o_ref[...] = o.reshape(o.shape[0], nt, td) + b2_ref[...]


def _mlp_body_alias(x_ref, w1_ref, b1_ref, w2_ref, b2_ref, _, o_ref):
    _mlp_body(x_ref, w1_ref, b1_ref, w2_ref, b2_ref, o_ref)


def _tc_mlp_chunk(x_chunk, W1, b1, W2, bias2, out_buf, B, blk0, blk):
    CB, D = x_chunk.shape
    H = W1.shape[1]
    O = W2.shape[1]
    NT = bias2.shape[0]
    TD = O // NT
    grid = (CB // blk,)
    in_specs = [
        pl.BlockSpec((blk, D), lambda i: (i, 0)),
        pl.BlockSpec((D, H), lambda i: (0, 0)),
        pl.BlockSpec((1, H), lambda i: (0, 0)),
        pl.BlockSpec((H, O), lambda i: (0, 0)),
        pl.BlockSpec((NT, TD), lambda i: (0, 0)),
    ]
    args = (x_chunk, W1, b1, W2, bias2)
    body = _mlp_body
    aliases = {}
    if out_buf is not None:
        in_specs.append(pl.BlockSpec(memory_space=pl.ANY))
        args = args + (out_buf,)
        body = _mlp_body_alias
        aliases = {5: 0}
    return pl.pallas_call(
        body,
        grid=grid,
        in_specs=in_specs,
        out_specs=pl.BlockSpec((blk, NT, TD), lambda i: (i + blk0, 0, 0)),
        out_shape=jax.ShapeDtypeStruct((B, NT, TD), jnp.float32),
        input_output_aliases=aliases,
    )(*args)


def kernel(labels, table, W1, b1, W2, b2, pos):
    B = labels.shape[0]
    V, D = table.shape
    NT, _ = pos.shape
    O = W2.shape[1]
    idx = labels.astype(jnp.int32)
    bias2 = b2.reshape(NT, D) + pos
    W1b = W1.astype(jnp.bfloat16)
    W2b = W2.astype(jnp.bfloat16)
    b1r = b1[None, :]

    BLK = 2048
    x = _make_sc_gather(V, D, B)(table, idx)
    return _tc_mlp_chunk(x, W1b, b1r, W2b, bias2, None, B, 0, BLK)
